# baseline (speedup 1.0000x reference)
import functools

import jax
import jax.numpy as jnp
from jax import lax
from jax.experimental import pallas as pl
from jax.experimental.pallas import tpu as pltpu
from jax.experimental.pallas import tpu_sc as plsc


@functools.lru_cache(maxsize=None)
def _make_gather(num_ids: int, dim: int):
    info = plsc.get_sparse_core_info()
    num_workers = info.num_cores * info.num_subcores
    assert num_ids % (8 * num_workers) == 0
    b_per_w = num_ids // num_workers

    mesh = plsc.VectorSubcoreMesh(core_axis_name="c", subcore_axis_name="s")

    @functools.partial(
        pl.kernel,
        mesh=mesh,
        out_type=jax.ShapeDtypeStruct((num_ids, dim), jnp.float32),
        scratch_types=[
            pltpu.VMEM((b_per_w,), jnp.int32),
            pltpu.VMEM((b_per_w, dim), jnp.float32),
            pltpu.SemaphoreType.DMA,
        ],
        compiler_params=pltpu.CompilerParams(use_tc_tiling_on_sc=False),
    )
    def gather_kernel(idx_hbm, table_hbm, out_hbm, idx_v, rows_v, sem):
        wid = lax.axis_index("s") * info.num_cores + lax.axis_index("c")
        base = wid * b_per_w
        pltpu.sync_copy(idx_hbm.at[pl.ds(base, b_per_w)], idx_v)
        pltpu.async_copy(table_hbm.at[idx_v], rows_v, sem).wait()
        pltpu.sync_copy(rows_v, out_hbm.at[pl.ds(base, b_per_w)])

    return gather_kernel


def kernel(input_ids, weight):
    ids = input_ids.astype(jnp.int32)
    fn = _make_gather(ids.shape[0], weight.shape[1])
    return fn(ids, weight)


# speedup vs baseline: 3.0110x; 3.0110x over previous
import functools

import jax
import jax.numpy as jnp
from jax import lax
from jax.experimental import pallas as pl
from jax.experimental.pallas import tpu as pltpu
from jax.experimental.pallas import tpu_sc as plsc

_RING = 8
_G = 16


@functools.lru_cache(maxsize=None)
def _make_colgather(num_ids: int, dim: int):
    info = plsc.get_sparse_core_info()
    num_workers = info.num_cores * info.num_subcores
    assert num_ids % (8 * num_workers) == 0
    b_per_w = num_ids // num_workers
    n_groups = b_per_w // _G

    mesh = plsc.VectorSubcoreMesh(core_axis_name="c", subcore_axis_name="s")

    @functools.partial(
        pl.kernel,
        mesh=mesh,
        out_type=jax.ShapeDtypeStruct((dim, num_ids), jnp.float32),
        scratch_types=[
            pltpu.VMEM((b_per_w,), jnp.int32),
            pltpu.VMEM((_RING, dim, 128), jnp.float32),
            pltpu.VMEM((dim, b_per_w), jnp.float32),
            [pltpu.SemaphoreType.DMA] * _RING,
        ],
        compiler_params=pltpu.CompilerParams(
            use_tc_tiling_on_sc=True, needs_layout_passes=False
        ),
    )
    def colgather(idx_hbm, wt_hbm, out_hbm, idx_v, ring, resbuf, sems):
        wid = lax.axis_index("s") * info.num_cores + lax.axis_index("c")
        base = wid * b_per_w
        pltpu.sync_copy(idx_hbm.at[pl.ds(base, b_per_w)], idx_v)

        lane = lax.iota(jnp.int32, 16)
        rows = [lane + 16 * k for k in range(dim // 16)]

        def fire(v, slot):
            col0 = pl.multiple_of(v & -128, 128)
            pltpu.async_copy(
                wt_hbm.at[:, pl.ds(col0, 128)], ring.at[slot], sems[slot]
            )

        def wait(slot):
            pltpu.make_async_copy(
                wt_hbm.at[:, pl.ds(0, 128)], ring.at[slot], sems[slot]
            ).wait()

        def extract(j, slot, v):
            c = jnp.broadcast_to(v & 127, (16,))
            jv = jnp.broadcast_to(j, (16,))
            for k in range(dim // 16):
                vals = plsc.load_gather(ring.at[slot], [rows[k], c])
                plsc.store_scatter(resbuf, [rows[k], jv], vals)

        vec0 = idx_v[pl.ds(0, _G)]
        for i in range(_RING):
            fire(vec0[i], i)

        def body(g, _):
            vec = idx_v[pl.ds(_G * g, _G)]
            vecn = idx_v[pl.ds(_G * g + _G, _G)]
            for i in range(_G):
                slot = i % _RING
                wait(slot)
                extract(_G * g + i, slot, vec[i])
                vnext = vec[i + _RING] if i < _RING else vecn[i - _RING]
                fire(vnext, slot)
            return 0

        lax.fori_loop(0, n_groups - 1, body, 0)

        vec = idx_v[pl.ds(_G * (n_groups - 1), _G)]
        for i in range(_G):
            slot = i % _RING
            wait(slot)
            extract(_G * (n_groups - 1) + i, slot, vec[i])
            if i + _RING < _G:
                fire(vec[i + _RING], slot)

        pltpu.sync_copy(resbuf, out_hbm.at[:, pl.ds(base, b_per_w)])

    return colgather


def kernel(input_ids, weight):
    ids = input_ids.astype(jnp.int32)
    fn = _make_colgather(ids.shape[0], weight.shape[1])
    out_t = fn(ids, weight.T)
    return out_t.T
